# baseline (speedup 1.0000x reference)
import dataclasses
import functools

import jax
import jax.numpy as jnp
import numpy as np
from jax import lax
from jax.experimental import pallas as pl
from jax.experimental.pallas import tpu as pltpu
from jax.experimental.pallas import tpu_sc as plsc

N = 50000
T = 256
E = 800000
G = 64
D = 64

EP = 819200
EROWS = EP // 128
PADROW = N
ACC_ROWS = 50048
HR = 391

HIST_ROWS_PER_TILE = EROWS // 32
SCAT_ROWS_PER_TILE = EROWS // 16
SCAT_CHUNK = 200
ZB_ROWS = 136
FQ = 16

BN_A = 2000
BN_B = 2000

@functools.cache
def _sc_mesh():
    return plsc.VectorSubcoreMesh(core_axis_name="c", subcore_axis_name="s")


def _sc_params():
    cp = pltpu.CompilerParams()
    fields = pltpu.CompilerParams.__dataclass_fields__
    if "needs_layout_passes" in fields:
        cp = dataclasses.replace(cp, needs_layout_passes=False)
    if "use_tc_tiling_on_sc" in fields:
        cp = dataclasses.replace(cp, use_tc_tiling_on_sc=False)
    return cp


@functools.cache
def _get_sc_hist():
  @functools.partial(
    pl.kernel,
    out_type=jax.ShapeDtypeStruct((32, HR, 128), jnp.float32),
    mesh=_sc_mesh(),
    compiler_params=_sc_params(),
    scratch_types=[
        pltpu.VMEM((HIST_ROWS_PER_TILE, 128), jnp.int32),
        pltpu.VMEM((HR, 128), jnp.float32),
    ],
  )
  def _sc_hist(dst_hbm, deg_hbm, idx_v, hist_v):
    c = lax.axis_index("c")
    s = lax.axis_index("s")
    w = c * 16 + s
    z16 = jnp.zeros((16,), jnp.float32)
    one16 = jnp.ones((16,), jnp.float32)

    @pl.loop(0, HR)
    def _(r):
        for k in range(8):
            hist_v[r, pl.ds(k * 16, 16)] = z16

    pltpu.sync_copy(dst_hbm.at[pl.ds(w * HIST_ROWS_PER_TILE, HIST_ROWS_PER_TILE)],
                    idx_v)

    @pl.loop(0, HIST_ROWS_PER_TILE)
    def _(r):
        for k in range(8):
            idx = idx_v[r, pl.ds(k * 16, 16)]
            plsc.addupdate_scatter(
                hist_v,
                [jax.lax.shift_right_logical(idx, 7), idx & 127],
                one16, mask=idx < N)

    pltpu.sync_copy(hist_v, deg_hbm.at[w])

  return _sc_hist


@functools.cache
def _get_sc_scatter():
  @functools.partial(
    pl.kernel,
    out_type=[jax.ShapeDtypeStruct((N, FQ), jnp.float32) for _ in range(4)],
    mesh=_sc_mesh(),
    compiler_params=_sc_params(),
    scratch_types=[
        pltpu.VMEM((SCAT_CHUNK, 128), jnp.int32),
        pltpu.VMEM((SCAT_CHUNK, 128), jnp.int32),
        pltpu.VMEM((8, 128, FQ), jnp.float32),
        pltpu.VMEM((ZB_ROWS, FQ), jnp.float32),
        pltpu.VMEM_SHARED((ACC_ROWS, FQ), jnp.float32),
    ] + [pltpu.SemaphoreType.DMA] * 16,
  )
  def _sc_scatter(u0_hbm, u1_hbm, u2_hbm, u3_hbm, src_hbm, dst_hbm,
                  acc0_hbm, acc1_hbm, acc2_hbm, acc3_hbm,
                  srcv, dstv, rowsb, zb_v, acc_sh, *sems):
    c = lax.axis_index("c")
    s = lax.axis_index("s")
    z16 = jnp.zeros((16,), jnp.float32)
    rows = [rowsb.at[b] for b in range(8)]
    gsem = sems[:8]
    ssem = sems[8:]

    def zero_acc():
        for t in range(23):
            pltpu.sync_copy(zb_v,
                            acc_sh.at[pl.ds(s * 3128 + t * ZB_ROWS, ZB_ROWS)])

    def edge_pass(u_hbm):
        for m in range(SCAT_ROWS_PER_TILE // SCAT_CHUNK):
            base = s * SCAT_ROWS_PER_TILE + m * SCAT_CHUNK
            pltpu.sync_copy(src_hbm.at[pl.ds(base, SCAT_CHUNK)], srcv)
            pltpu.sync_copy(dst_hbm.at[pl.ds(base, SCAT_CHUNK)], dstv)
            for b in range(4):
                pltpu.async_copy(u_hbm.at[srcv.at[b]], rows[b], gsem[b])

            @pl.loop(0, SCAT_CHUNK // 8)
            def _(i):
                for b in range(8):
                    r = i * 8 + b
                    bp = (b + 4) % 8

                    @pl.when(r >= 4)
                    def _():
                        pltpu.make_async_copy(
                            rows[bp], acc_sh.at[dstv.at[r]],
                            ssem[bp]).wait()

                    @pl.when(r + 4 < SCAT_CHUNK)
                    def _():
                        pltpu.async_copy(u_hbm.at[srcv.at[r + 4]],
                                         rows[bp], gsem[bp])

                    pltpu.make_async_copy(u_hbm.at[srcv.at[r]],
                                          rows[b], gsem[b]).wait()
                    pltpu.async_copy(rows[b], acc_sh.at[dstv.at[r]],
                                     ssem[b], add=True)

            for b in range(4, 8):
                pltpu.make_async_copy(rows[b], acc_sh.at[dstv.at[b]],
                                      ssem[b]).wait()

    def writeout(acc_hbm):
        @pl.when(s < 15)
        def _():
            pltpu.sync_copy(acc_sh.at[pl.ds(s * 3128, 3128)],
                            acc_hbm.at[pl.ds(s * 3128, 3128)])

        @pl.when(s == 15)
        def _():
            pltpu.sync_copy(acc_sh.at[pl.ds(15 * 3128, 3080)],
                            acc_hbm.at[pl.ds(15 * 3128, 3080)])

    @pl.loop(0, ZB_ROWS)
    def _(r):
        zb_v[r, pl.ds(0, 16)] = z16

    zero_acc()
    plsc.subcore_barrier()

    @pl.when(c == 0)
    def _():
        edge_pass(u0_hbm)

    @pl.when(c == 1)
    def _():
        edge_pass(u2_hbm)

    plsc.subcore_barrier()

    @pl.when(c == 0)
    def _():
        writeout(acc0_hbm)

    @pl.when(c == 1)
    def _():
        writeout(acc2_hbm)

    zero_acc()
    plsc.subcore_barrier()

    @pl.when(c == 0)
    def _():
        edge_pass(u1_hbm)

    @pl.when(c == 1)
    def _():
        edge_pass(u3_hbm)

    plsc.subcore_barrier()

    @pl.when(c == 0)
    def _():
        writeout(acc1_hbm)

    @pl.when(c == 1)
    def _():
        writeout(acc3_hbm)

  return _sc_scatter


def _tc_a_body(x_ref, deg_ref, c1_ref, c2_ref, bb1_ref, bb2_ref,
               bnsc_ref, bnsh_ref, gw_ref,
               u0_ref, u1_ref, u2_ref, u3_ref):
    bf = jnp.bfloat16
    xb = x_ref[...].astype(bf)
    h1 = jnp.dot(xb, c1_ref[0].astype(bf), preferred_element_type=jnp.float32)
    for k in range(1, 4):
        h1 = jnp.maximum(h1, jnp.dot(xb, c1_ref[k].astype(bf),
                                     preferred_element_type=jnp.float32))
    h1 = (h1 + bb1_ref[...]).astype(bf)
    h2 = jnp.dot(h1, c2_ref[0].astype(bf), preferred_element_type=jnp.float32)
    for k in range(1, 4):
        h2 = jnp.maximum(h2, jnp.dot(h1, c2_ref[k].astype(bf),
                                     preferred_element_type=jnp.float32))
    h2 = h2 + bb2_ref[...]
    h = jnp.maximum(h2 * bnsc_ref[...] + bnsh_ref[...], 0.0)
    hw = jnp.dot(h.astype(bf), gw_ref[...].astype(bf),
                 preferred_element_type=jnp.float32)
    deg_col = jnp.dot(deg_ref[...], jnp.ones((32, 1), jnp.float32),
                      preferred_element_type=jnp.float32)
    dinv = lax.rsqrt(deg_col + 1.0)
    u = hw * dinv
    u0_ref[...] = u[:, 0:16]
    u1_ref[...] = u[:, 16:32]
    u2_ref[...] = u[:, 32:48]
    u3_ref[...] = u[:, 48:64]


def _make_tc_a(interpret=False):
  return pl.pallas_call(
    _tc_a_body,
    interpret=interpret,
    grid=(N // BN_A,),
    in_specs=[
        pl.BlockSpec((BN_A, T), lambda i: (i, 0)),
        pl.BlockSpec((BN_A, 32), lambda i: (i, 0)),
        pl.BlockSpec((4, T, 128), lambda i: (0, 0, 0)),
        pl.BlockSpec((4, 128, 64), lambda i: (0, 0, 0)),
        pl.BlockSpec((1, 128), lambda i: (0, 0)),
        pl.BlockSpec((1, 64), lambda i: (0, 0)),
        pl.BlockSpec((1, 64), lambda i: (0, 0)),
        pl.BlockSpec((1, 64), lambda i: (0, 0)),
        pl.BlockSpec((64, 64), lambda i: (0, 0)),
    ],
    out_specs=[pl.BlockSpec((BN_A, FQ), lambda i: (i, 0)) for _ in range(4)],
    out_shape=[jax.ShapeDtypeStruct((N, FQ), jnp.float32) for _ in range(4)],
  )


_tc_a = _make_tc_a()


def _tc_b_body(acc0_ref, acc1_ref, acc2_ref, acc3_ref,
               u0_ref, u1_ref, u2_ref, u3_ref, deg_ref, b_ref,
               batch_ref, lw_ref, lb_ref, out_ref, sums_s, cnt_s):
    j = pl.program_id(0)
    a = jnp.concatenate([acc0_ref[...] + u0_ref[...],
                         acc1_ref[...] + u1_ref[...],
                         acc2_ref[...] + u2_ref[...],
                         acc3_ref[...] + u3_ref[...]], axis=1)
    deg_col = jnp.dot(deg_ref[...], jnp.ones((32, 1), jnp.float32),
                      preferred_element_type=jnp.float32)
    dinv = lax.rsqrt(deg_col + 1.0)
    y = jnp.maximum(a * dinv + b_ref[...], 0.0)
    onehot = (batch_ref[...] ==
              lax.broadcasted_iota(jnp.int32, (BN_B, G), 1)).astype(jnp.float32)
    psum = lax.dot_general(onehot, y, (((0,), (0,)), ((), ())),
                           preferred_element_type=jnp.float32)
    pcnt = lax.dot_general(onehot, jnp.ones((BN_B, 1), jnp.float32),
                           (((0,), (0,)), ((), ())),
                           preferred_element_type=jnp.float32)

    @pl.when(j == 0)
    def _():
        sums_s[...] = jnp.zeros_like(sums_s)
        cnt_s[...] = jnp.zeros_like(cnt_s)

    sums_s[...] += psum
    cnt_s[...] += pcnt

    @pl.when(j == pl.num_programs(0) - 1)
    def _():
        mean = sums_s[...] / jnp.maximum(cnt_s[...], 1.0)
        logits = jnp.dot(mean, lw_ref[...],
                         preferred_element_type=jnp.float32) + lb_ref[...]
        out_ref[...] = jax.nn.sigmoid(logits)


def _make_tc_b(interpret=False):
  return pl.pallas_call(
    _tc_b_body,
    interpret=interpret,
    grid=(N // BN_B,),
    in_specs=[pl.BlockSpec((BN_B, FQ), lambda j: (j, 0)) for _ in range(8)]
             + [
        pl.BlockSpec((BN_B, 32), lambda j: (j, 0)),
        pl.BlockSpec((1, G), lambda j: (0, 0)),
        pl.BlockSpec((BN_B, 1), lambda j: (j, 0)),
        pl.BlockSpec((G, 1), lambda j: (0, 0)),
        pl.BlockSpec((1, 1), lambda j: (0, 0)),
    ],
    out_specs=pl.BlockSpec((G, 1), lambda j: (0, 0)),
    out_shape=jax.ShapeDtypeStruct((G, 1), jnp.float32),
    scratch_shapes=[
        pltpu.VMEM((G, G), jnp.float32),
        pltpu.VMEM((G, 1), jnp.float32),
    ],
  )


_tc_b = _make_tc_b()


def _phase_mats(conv1_w, conv2_w):
    k_ = np.arange(4)[:, None, None, None]
    j1 = np.arange(3)[None, :, None, None]
    s1 = np.arange(T)[None, None, :, None]
    t1 = np.arange(64)[None, None, None, :]
    p1 = (s1 == 4 * t1 + k_ + j1 - 1).astype(np.float32)
    c1 = jnp.einsum('cj,kjst->kstc', conv1_w[:, 0, :], jnp.asarray(p1))
    c1 = jnp.transpose(c1, (0, 1, 3, 2)).reshape(4, T, 128)
    j2 = np.arange(5)[None, :, None, None]
    s2 = np.arange(64)[None, None, :, None]
    t2 = np.arange(16)[None, None, None, :]
    p2 = (s2 == 4 * t2 + k_ + j2 - 2).astype(np.float32)
    c2 = jnp.einsum('oij,kjst->kisot', conv2_w, jnp.asarray(p2))
    c2 = c2.reshape(4, 128, 64)
    return c1, c2


def kernel(x, edge_index, batch, conv1_w, conv1_b, conv2_w, conv2_b,
           bn_gamma, bn_beta, bn_rm, bn_rv, gcn_w, gcn_b, lin_w, lin_b):
    src = edge_index[0]
    dst = edge_index[1]
    pad = EP - E
    src_p = jnp.concatenate(
        [src, jnp.zeros((pad,), jnp.int32)]).reshape(EROWS, 128)
    dst_p = jnp.concatenate(
        [dst, jnp.full((pad,), PADROW, jnp.int32)]).reshape(EROWS, 128)

    c1, c2 = _phase_mats(conv1_w, conv2_w)
    bb1 = jnp.repeat(conv1_b, 64).reshape(1, 128)
    bb2 = jnp.repeat(conv2_b, 16).reshape(1, 64)
    bnsc = (bn_gamma * lax.rsqrt(bn_rv + 1e-5)).reshape(1, D)
    bnsh = (bn_beta - bn_rm * bn_gamma * lax.rsqrt(bn_rv + 1e-5)).reshape(1, D)

    deg_t = _get_sc_hist()(dst_p).reshape(32, HR * 128)[:, :N].T
    u0, u1, u2, u3 = _tc_a(x, deg_t, c1, c2, bb1, bb2, bnsc, bnsh, gcn_w)
    acc0, acc1, acc2, acc3 = _get_sc_scatter()(u0, u1, u2, u3, src_p, dst_p)
    out = _tc_b(acc0, acc1, acc2, acc3, u0, u1, u2, u3, deg_t,
                gcn_b.reshape(1, D), batch.reshape(N, 1), lin_w,
                lin_b.reshape(1, 1))
    return out


# speedup vs baseline: 1.0280x; 1.0280x over previous
import dataclasses
import functools

import jax
import jax.numpy as jnp
import numpy as np
from jax import lax
from jax.experimental import pallas as pl
from jax.experimental.pallas import tpu as pltpu
from jax.experimental.pallas import tpu_sc as plsc

N = 50000
T = 256
E = 800000
G = 64
D = 64

EP = 819200
EROWS = EP // 128
PADROW = N
ACC_ROWS = 50048
HR = 391

HIST_ROWS_PER_TILE = EROWS // 32
SCAT_ROWS_PER_TILE = EROWS // 16
SCAT_CHUNK = 200
ZB_ROWS = 136
FQ = 16

BN_A = 2000
BN_B = 2000

@functools.cache
def _sc_mesh():
    return plsc.VectorSubcoreMesh(core_axis_name="c", subcore_axis_name="s")


def _sc_params():
    cp = pltpu.CompilerParams()
    fields = pltpu.CompilerParams.__dataclass_fields__
    if "needs_layout_passes" in fields:
        cp = dataclasses.replace(cp, needs_layout_passes=False)
    if "use_tc_tiling_on_sc" in fields:
        cp = dataclasses.replace(cp, use_tc_tiling_on_sc=False)
    return cp


@functools.cache
def _get_sc_hist():
  @functools.partial(
    pl.kernel,
    out_type=jax.ShapeDtypeStruct((32, HR, 128), jnp.float32),
    mesh=_sc_mesh(),
    compiler_params=_sc_params(),
    scratch_types=[
        pltpu.VMEM((HIST_ROWS_PER_TILE, 128), jnp.int32),
        pltpu.VMEM((HR, 128), jnp.float32),
    ],
  )
  def _sc_hist(dst_hbm, deg_hbm, idx_v, hist_v):
    c = lax.axis_index("c")
    s = lax.axis_index("s")
    w = c * 16 + s
    z16 = jnp.zeros((16,), jnp.float32)
    one16 = jnp.ones((16,), jnp.float32)

    @pl.loop(0, HR)
    def _(r):
        for k in range(8):
            hist_v[r, pl.ds(k * 16, 16)] = z16

    pltpu.sync_copy(dst_hbm.at[pl.ds(w * HIST_ROWS_PER_TILE, HIST_ROWS_PER_TILE)],
                    idx_v)

    @pl.loop(0, HIST_ROWS_PER_TILE)
    def _(r):
        for k in range(8):
            idx = idx_v[r, pl.ds(k * 16, 16)]
            plsc.addupdate_scatter(
                hist_v,
                [jax.lax.shift_right_logical(idx, 7), idx & 127],
                one16, mask=idx < N)

    pltpu.sync_copy(hist_v, deg_hbm.at[w])

  return _sc_hist


@functools.cache
def _get_sc_scatter():
  @functools.partial(
    pl.kernel,
    out_type=[jax.ShapeDtypeStruct((N, FQ), jnp.float32) for _ in range(4)],
    mesh=_sc_mesh(),
    compiler_params=_sc_params(),
    scratch_types=[
        pltpu.VMEM((SCAT_CHUNK, 128), jnp.int32),
        pltpu.VMEM((SCAT_CHUNK, 128), jnp.int32),
        pltpu.VMEM((8, 128, FQ), jnp.float32),
        pltpu.VMEM((ZB_ROWS, FQ), jnp.float32),
        pltpu.VMEM_SHARED((ACC_ROWS, FQ), jnp.float32),
    ] + [pltpu.SemaphoreType.DMA] * 16,
  )
  def _sc_scatter(u0_hbm, u1_hbm, u2_hbm, u3_hbm, src_hbm, dst_hbm,
                  acc0_hbm, acc1_hbm, acc2_hbm, acc3_hbm,
                  srcv, dstv, rowsb, zb_v, acc_sh, *sems):
    c = lax.axis_index("c")
    s = lax.axis_index("s")
    z16 = jnp.zeros((16,), jnp.float32)
    rows = [rowsb.at[b] for b in range(8)]
    gsem = sems[:8]
    ssem = sems[8:]

    def zero_acc():
        for t in range(23):
            pltpu.sync_copy(zb_v,
                            acc_sh.at[pl.ds(s * 3128 + t * ZB_ROWS, ZB_ROWS)])

    def edge_pass(u_hbm):
        for m in range(SCAT_ROWS_PER_TILE // SCAT_CHUNK):
            base = s * SCAT_ROWS_PER_TILE + m * SCAT_CHUNK
            pltpu.sync_copy(src_hbm.at[pl.ds(base, SCAT_CHUNK)], srcv)
            pltpu.sync_copy(dst_hbm.at[pl.ds(base, SCAT_CHUNK)], dstv)
            for b in range(4):
                pltpu.async_copy(u_hbm.at[srcv.at[b]], rows[b], gsem[b])

            @pl.loop(0, SCAT_CHUNK // 8)
            def _(i):
                for b in range(8):
                    r = i * 8 + b
                    bp = (b + 4) % 8

                    @pl.when(r >= 4)
                    def _():
                        pltpu.make_async_copy(
                            rows[bp], acc_sh.at[dstv.at[r]],
                            ssem[bp]).wait()

                    @pl.when(r + 4 < SCAT_CHUNK)
                    def _():
                        pltpu.async_copy(u_hbm.at[srcv.at[r + 4]],
                                         rows[bp], gsem[bp])

                    pltpu.make_async_copy(u_hbm.at[srcv.at[r]],
                                          rows[b], gsem[b]).wait()
                    pltpu.async_copy(rows[b], acc_sh.at[dstv.at[r]],
                                     ssem[b], add=True)

            for b in range(4, 8):
                pltpu.make_async_copy(rows[b], acc_sh.at[dstv.at[b]],
                                      ssem[b]).wait()

    def writeout(acc_hbm):
        @pl.when(s < 15)
        def _():
            pltpu.sync_copy(acc_sh.at[pl.ds(s * 3128, 3128)],
                            acc_hbm.at[pl.ds(s * 3128, 3128)])

        @pl.when(s == 15)
        def _():
            pltpu.sync_copy(acc_sh.at[pl.ds(15 * 3128, 3080)],
                            acc_hbm.at[pl.ds(15 * 3128, 3080)])

    @pl.loop(0, ZB_ROWS)
    def _(r):
        zb_v[r, pl.ds(0, 16)] = z16

    zero_acc()
    plsc.subcore_barrier()

    @pl.when(c == 0)
    def _():
        edge_pass(u0_hbm)

    @pl.when(c == 1)
    def _():
        edge_pass(u2_hbm)

    plsc.subcore_barrier()

    @pl.when(c == 0)
    def _():
        writeout(acc0_hbm)

    @pl.when(c == 1)
    def _():
        writeout(acc2_hbm)

    zero_acc()
    plsc.subcore_barrier()

    @pl.when(c == 0)
    def _():
        edge_pass(u1_hbm)

    @pl.when(c == 1)
    def _():
        edge_pass(u3_hbm)

    plsc.subcore_barrier()

    @pl.when(c == 0)
    def _():
        writeout(acc1_hbm)

    @pl.when(c == 1)
    def _():
        writeout(acc3_hbm)

  return _sc_scatter


def _tc_a_body(x_ref, deg_ref, c1_ref, c2_ref, bb1_ref, bb2_ref,
               bnsc_ref, bnsh_ref, gw_ref,
               u0_ref, u1_ref, u2_ref, u3_ref, dinv_ref):
    xb = x_ref[...]
    h1 = jnp.dot(xb, c1_ref[0], preferred_element_type=jnp.float32)
    for k in range(1, 4):
        h1 = jnp.maximum(h1, jnp.dot(xb, c1_ref[k],
                                     preferred_element_type=jnp.float32))
    h1 = h1 + bb1_ref[...]
    h2 = jnp.dot(h1, c2_ref[0], preferred_element_type=jnp.float32)
    for k in range(1, 4):
        h2 = jnp.maximum(h2, jnp.dot(h1, c2_ref[k],
                                     preferred_element_type=jnp.float32))
    h2 = h2 + bb2_ref[...]
    h = jnp.maximum(h2 * bnsc_ref[...] + bnsh_ref[...], 0.0)
    hw = jnp.dot(h, gw_ref[...], preferred_element_type=jnp.float32)
    deg_col = jnp.dot(deg_ref[...], jnp.ones((32, 1), jnp.float32),
                      preferred_element_type=jnp.float32)
    dinv = lax.rsqrt(deg_col + 1.0)
    u = hw * dinv
    u0_ref[...] = u[:, 0:16]
    u1_ref[...] = u[:, 16:32]
    u2_ref[...] = u[:, 32:48]
    u3_ref[...] = u[:, 48:64]
    dinv_ref[...] = dinv


def _make_tc_a(interpret=False):
  return pl.pallas_call(
    _tc_a_body,
    interpret=interpret,
    grid=(N // BN_A,),
    in_specs=[
        pl.BlockSpec((BN_A, T), lambda i: (i, 0)),
        pl.BlockSpec((BN_A, 32), lambda i: (i, 0)),
        pl.BlockSpec((4, T, 128), lambda i: (0, 0, 0)),
        pl.BlockSpec((4, 128, 64), lambda i: (0, 0, 0)),
        pl.BlockSpec((1, 128), lambda i: (0, 0)),
        pl.BlockSpec((1, 64), lambda i: (0, 0)),
        pl.BlockSpec((1, 64), lambda i: (0, 0)),
        pl.BlockSpec((1, 64), lambda i: (0, 0)),
        pl.BlockSpec((64, 64), lambda i: (0, 0)),
    ],
    out_specs=[pl.BlockSpec((BN_A, FQ), lambda i: (i, 0)) for _ in range(4)]
              + [pl.BlockSpec((BN_A, 1), lambda i: (i, 0))],
    out_shape=[jax.ShapeDtypeStruct((N, FQ), jnp.float32) for _ in range(4)]
              + [jax.ShapeDtypeStruct((N, 1), jnp.float32)],
  )


_tc_a = _make_tc_a()


def _tc_b_body(acc0_ref, acc1_ref, acc2_ref, acc3_ref,
               u0_ref, u1_ref, u2_ref, u3_ref, dinv_ref, b_ref,
               batch_ref, lw_ref, lb_ref, out_ref, sums_s, cnt_s):
    j = pl.program_id(0)
    a = jnp.concatenate([acc0_ref[...] + u0_ref[...],
                         acc1_ref[...] + u1_ref[...],
                         acc2_ref[...] + u2_ref[...],
                         acc3_ref[...] + u3_ref[...]], axis=1)
    y = jnp.maximum(a * dinv_ref[...] + b_ref[...], 0.0)
    onehot = (batch_ref[...] ==
              lax.broadcasted_iota(jnp.int32, (BN_B, G), 1)).astype(jnp.float32)
    psum = lax.dot_general(onehot, y, (((0,), (0,)), ((), ())),
                           preferred_element_type=jnp.float32)
    pcnt = lax.dot_general(onehot, jnp.ones((BN_B, 1), jnp.float32),
                           (((0,), (0,)), ((), ())),
                           preferred_element_type=jnp.float32)

    @pl.when(j == 0)
    def _():
        sums_s[...] = jnp.zeros_like(sums_s)
        cnt_s[...] = jnp.zeros_like(cnt_s)

    sums_s[...] += psum
    cnt_s[...] += pcnt

    @pl.when(j == pl.num_programs(0) - 1)
    def _():
        mean = sums_s[...] / jnp.maximum(cnt_s[...], 1.0)
        logits = jnp.dot(mean, lw_ref[...],
                         preferred_element_type=jnp.float32) + lb_ref[...]
        out_ref[...] = jax.nn.sigmoid(logits)


def _make_tc_b(interpret=False):
  return pl.pallas_call(
    _tc_b_body,
    interpret=interpret,
    grid=(N // BN_B,),
    in_specs=[pl.BlockSpec((BN_B, FQ), lambda j: (j, 0)) for _ in range(8)]
             + [
        pl.BlockSpec((BN_B, 1), lambda j: (j, 0)),
        pl.BlockSpec((1, G), lambda j: (0, 0)),
        pl.BlockSpec((BN_B, 1), lambda j: (j, 0)),
        pl.BlockSpec((G, 1), lambda j: (0, 0)),
        pl.BlockSpec((1, 1), lambda j: (0, 0)),
    ],
    out_specs=pl.BlockSpec((G, 1), lambda j: (0, 0)),
    out_shape=jax.ShapeDtypeStruct((G, 1), jnp.float32),
    scratch_shapes=[
        pltpu.VMEM((G, G), jnp.float32),
        pltpu.VMEM((G, 1), jnp.float32),
    ],
  )


_tc_b = _make_tc_b()


def _phase_mats(conv1_w, conv2_w):
    k_ = np.arange(4)[:, None, None, None]
    j1 = np.arange(3)[None, :, None, None]
    s1 = np.arange(T)[None, None, :, None]
    t1 = np.arange(64)[None, None, None, :]
    p1 = (s1 == 4 * t1 + k_ + j1 - 1).astype(np.float32)
    c1 = jnp.einsum('cj,kjst->kstc', conv1_w[:, 0, :], jnp.asarray(p1))
    c1 = jnp.transpose(c1, (0, 1, 3, 2)).reshape(4, T, 128)
    j2 = np.arange(5)[None, :, None, None]
    s2 = np.arange(64)[None, None, :, None]
    t2 = np.arange(16)[None, None, None, :]
    p2 = (s2 == 4 * t2 + k_ + j2 - 2).astype(np.float32)
    c2 = jnp.einsum('oij,kjst->kisot', conv2_w, jnp.asarray(p2))
    c2 = c2.reshape(4, 128, 64)
    return c1, c2


def kernel(x, edge_index, batch, conv1_w, conv1_b, conv2_w, conv2_b,
           bn_gamma, bn_beta, bn_rm, bn_rv, gcn_w, gcn_b, lin_w, lin_b):
    src = edge_index[0]
    dst = edge_index[1]
    pad = EP - E
    src_p = jnp.concatenate(
        [src, jnp.zeros((pad,), jnp.int32)]).reshape(EROWS, 128)
    dst_p = jnp.concatenate(
        [dst, jnp.full((pad,), PADROW, jnp.int32)]).reshape(EROWS, 128)

    c1, c2 = _phase_mats(conv1_w, conv2_w)
    bb1 = jnp.repeat(conv1_b, 64).reshape(1, 128)
    bb2 = jnp.repeat(conv2_b, 16).reshape(1, 64)
    bnsc = (bn_gamma * lax.rsqrt(bn_rv + 1e-5)).reshape(1, D)
    bnsh = (bn_beta - bn_rm * bn_gamma * lax.rsqrt(bn_rv + 1e-5)).reshape(1, D)

    deg_t = _get_sc_hist()(dst_p).reshape(32, HR * 128)[:, :N].T
    u0, u1, u2, u3, dinv = _tc_a(x, deg_t, c1, c2, bb1, bb2, bnsc, bnsh, gcn_w)
    acc0, acc1, acc2, acc3 = _get_sc_scatter()(u0, u1, u2, u3, src_p, dst_p)
    out = _tc_b(acc0, acc1, acc2, acc3, u0, u1, u2, u3, dinv,
                gcn_b.reshape(1, D), batch.reshape(N, 1), lin_w,
                lin_b.reshape(1, 1))
    return out
